# baseline (speedup 1.0000x reference)
import functools

import jax
import jax.numpy as jnp
from jax.experimental import pallas as pl


def _linear_kernel(x_ref, w_ref, b_ref, o_ref, *, relu):
    acc = jnp.dot(x_ref[...], w_ref[...], preferred_element_type=jnp.float32)
    acc = acc + b_ref[...]
    if relu:
        acc = jnp.maximum(acc, 0.0)
    o_ref[...] = acc


def _linear(x, W, b, relu, bn=512):
    M, K = x.shape
    _, N = W.shape
    bn = min(bn, N)
    while N % bn:
        bn -= 128
    grid = (N // bn,)
    return pl.pallas_call(
        functools.partial(_linear_kernel, relu=relu),
        grid=grid,
        in_specs=[
            pl.BlockSpec((M, K), lambda j: (0, 0)),
            pl.BlockSpec((K, bn), lambda j: (0, j)),
            pl.BlockSpec((1, bn), lambda j: (0, j)),
        ],
        out_specs=pl.BlockSpec((M, bn), lambda j: (0, j)),
        out_shape=jax.ShapeDtypeStruct((M, N), jnp.float32),
    )(x, W, b.reshape(1, N))


def _mlp(params, x):
    n = len(params)
    for i, (W, b) in enumerate(params):
        x = _linear(x, W, b, relu=(i < n - 1))
    return x


def _rq_kernel(e_ref, cb_ref, zq_ref, idx_ref, loss_ref, *, n_layers):
    r = e_ref[...]
    zq = jnp.zeros_like(r)
    lsum = jnp.float32(0.0)
    idx_rows = []
    for i in range(n_layers):
        cb = cb_ref[i]
        d = (jnp.sum(r * r, axis=1, keepdims=True)
             - 2.0 * jax.lax.dot_general(
                 r, cb, (((1,), (1,)), ((), ())),
                 preferred_element_type=jnp.float32)
             + jnp.sum(cb * cb, axis=1)[None, :])
        idx = jnp.argmin(d, axis=1)
        onehot = (idx[:, None] == jax.lax.broadcasted_iota(
            jnp.int32, d.shape, 1)).astype(jnp.float32)
        q = jnp.dot(onehot, cb, preferred_element_type=jnp.float32,
                    precision=jax.lax.Precision.HIGHEST)
        lsum = lsum + jnp.sum((q - r) ** 2)
        q_st = r + (q - r)
        zq = zq + q_st
        r = r - q_st
        idx_rows.append(idx)
    zq_ref[...] = zq
    idx_ref[...] = jnp.stack(idx_rows, axis=0)
    loss_ref[...] = jnp.reshape(lsum, (1, 1))


def _rq(e, cb):
    B, E = e.shape
    L = cb.shape[0]
    zq, idxT, lsum = pl.pallas_call(
        functools.partial(_rq_kernel, n_layers=L),
        out_shape=(
            jax.ShapeDtypeStruct((B, E), jnp.float32),
            jax.ShapeDtypeStruct((L, B), jnp.int32),
            jax.ShapeDtypeStruct((1, 1), jnp.float32),
        ),
    )(e, cb)
    loss = 1.25 * lsum[0, 0] / (B * E)
    return zq, loss, idxT.T


def kernel(text_x, image_x, t_enc, t_dec, i_enc, i_dec, t_cb, i_cb):
    text_e = _mlp(t_enc, text_x)
    z_q_text, text_rq_loss, text_indices = _rq(text_e, t_cb)
    text_out = _mlp(t_dec, z_q_text)
    image_e = _mlp(i_enc, image_x)
    z_q_image, image_rq_loss, image_indices = _rq(image_e, i_cb)
    image_out = _mlp(i_dec, z_q_image)
    return (text_out, image_out, text_rq_loss, image_rq_loss,
            text_indices, image_indices, z_q_text, z_q_image)


# speedup vs baseline: 1.1862x; 1.1862x over previous
import functools

import jax
import jax.numpy as jnp
from jax.experimental import pallas as pl


def _linear_kernel(x_ref, w_ref, b_ref, o_ref, *, relu):
    acc = jnp.dot(x_ref[...], w_ref[...], preferred_element_type=jnp.float32)
    acc = acc + b_ref[...]
    if relu:
        acc = jnp.maximum(acc, 0.0)
    o_ref[...] = acc


def _linear(x, W, b, relu, bn=512):
    M, K = x.shape
    _, N = W.shape
    bn = min(bn, N)
    while N % bn:
        bn -= 128
    grid = (N // bn,)
    return pl.pallas_call(
        functools.partial(_linear_kernel, relu=relu),
        grid=grid,
        in_specs=[
            pl.BlockSpec((M, K), lambda j: (0, 0)),
            pl.BlockSpec((K, bn), lambda j: (0, j)),
            pl.BlockSpec((1, bn), lambda j: (0, j)),
        ],
        out_specs=pl.BlockSpec((M, bn), lambda j: (0, j)),
        out_shape=jax.ShapeDtypeStruct((M, N), jnp.float32),
    )(x, W, b.reshape(1, N))


def _rq_body(r, cb_ref, n_layers):
    zq = jnp.zeros_like(r)
    lsum = jnp.float32(0.0)
    idx_rows = []
    for i in range(n_layers):
        cb = cb_ref[i]
        d = (jnp.sum(r * r, axis=1, keepdims=True)
             - 2.0 * jax.lax.dot_general(
                 r, cb, (((1,), (1,)), ((), ())),
                 preferred_element_type=jnp.float32)
             + jnp.sum(cb * cb, axis=1)[None, :])
        idx = jnp.argmin(d, axis=1)
        onehot = (idx[:, None] == jax.lax.broadcasted_iota(
            jnp.int32, d.shape, 1)).astype(jnp.float32)
        q = jnp.dot(onehot, cb, preferred_element_type=jnp.float32,
                    precision=jax.lax.Precision.HIGHEST)
        lsum = lsum + jnp.sum((q - r) ** 2)
        q_st = r + (q - r)
        zq = zq + q_st
        r = r - q_st
        idx_rows.append(idx)
    return zq, jnp.stack(idx_rows, axis=0), lsum


def _mid_kernel(h_ref, w2_ref, b2_ref, w3_ref, b3_ref, w4_ref, b4_ref,
                cb_ref, dw1_ref, db1_ref, dw2_ref, db2_ref, dw3_ref, db3_ref,
                out_ref, zq_ref, idx_ref, loss_ref, *, n_layers):
    h = h_ref[...]
    h = jnp.maximum(jnp.dot(h, w2_ref[...],
                            preferred_element_type=jnp.float32) + b2_ref[...],
                    0.0)
    h = jnp.maximum(jnp.dot(h, w3_ref[...],
                            preferred_element_type=jnp.float32) + b3_ref[...],
                    0.0)
    e = jnp.dot(h, w4_ref[...],
                preferred_element_type=jnp.float32) + b4_ref[...]
    zq, idx, lsum = _rq_body(e, cb_ref, n_layers)
    zq_ref[...] = zq
    idx_ref[...] = idx
    loss_ref[...] = jnp.reshape(lsum, (1, 1))
    g = jnp.maximum(jnp.dot(zq, dw1_ref[...],
                            preferred_element_type=jnp.float32) + db1_ref[...],
                    0.0)
    g = jnp.maximum(jnp.dot(g, dw2_ref[...],
                            preferred_element_type=jnp.float32) + db2_ref[...],
                    0.0)
    g = jnp.maximum(jnp.dot(g, dw3_ref[...],
                            preferred_element_type=jnp.float32) + db3_ref[...],
                    0.0)
    out_ref[...] = g


def _mid(h1, enc, cb, dec):
    B = h1.shape[0]
    L = cb.shape[0]
    E = cb.shape[2]
    (W2, b2), (W3, b3), (W4, b4) = enc[1], enc[2], enc[3]
    (dW1, db1), (dW2, db2), (dW3, db3) = dec[0], dec[1], dec[2]
    N_out = dW3.shape[1]
    out, zq, idxT, lsum = pl.pallas_call(
        functools.partial(_mid_kernel, n_layers=L),
        out_shape=(
            jax.ShapeDtypeStruct((B, N_out), jnp.float32),
            jax.ShapeDtypeStruct((B, E), jnp.float32),
            jax.ShapeDtypeStruct((L, B), jnp.int32),
            jax.ShapeDtypeStruct((1, 1), jnp.float32),
        ),
    )(h1, W2, b2.reshape(1, -1), W3, b3.reshape(1, -1), W4, b4.reshape(1, -1),
      cb, dW1, db1.reshape(1, -1), dW2, db2.reshape(1, -1),
      dW3, db3.reshape(1, -1))
    loss = 1.25 * lsum[0, 0] / (B * E)
    return out, zq, loss, idxT.T


def _modality(x, enc, dec, cb):
    W1, b1 = enc[0]
    h1 = _linear(x, W1, b1, relu=True)
    h3, zq, loss, indices = _mid(h1, enc, cb, dec)
    dW4, db4 = dec[3]
    out = _linear(h3, dW4, db4, relu=False)
    return out, loss, indices, zq


def kernel(text_x, image_x, t_enc, t_dec, i_enc, i_dec, t_cb, i_cb):
    text_out, text_rq_loss, text_indices, z_q_text = _modality(
        text_x, t_enc, t_dec, t_cb)
    image_out, image_rq_loss, image_indices, z_q_image = _modality(
        image_x, i_enc, i_dec, i_cb)
    return (text_out, image_out, text_rq_loss, image_rq_loss,
            text_indices, image_indices, z_q_text, z_q_image)
